# baseline (speedup 1.0000x reference)
import functools

import jax
import jax.numpy as jnp
from jax import lax
from jax.experimental import pallas as pl
from jax.experimental.pallas import tpu as pltpu
from jax.experimental.pallas import tpu_sc as plsc

_B, _L, _D = 4096, 200, 64
_N = _B * _L
_NW = 32
_GROUP = 128
_SUPER = 4
_GROUPS_W = _N // (_NW * _GROUP)
_ITERS = _GROUPS_W // _SUPER


def _build():
    mesh = plsc.VectorSubcoreMesh(core_axis_name="c", subcore_axis_name="s")

    @functools.partial(
        pl.kernel,
        mesh=mesh,
        out_type=jax.ShapeDtypeStruct((_N // _GROUP, _GROUP, _D), jnp.float32),
        compiler_params=pltpu.CompilerParams(use_tc_tiling_on_sc=False),
        scratch_types=[
            pltpu.VMEM((_GROUPS_W, _GROUP), jnp.int32),
            pltpu.VMEM((_SUPER, _GROUP, _D), jnp.float32),
            pltpu.SemaphoreType.DMA,
        ],
    )
    def gather_kernel(table_hbm, idx_hbm, out_hbm, idx_v, buf, gsem):
        wid = lax.axis_index("s") * 2 + lax.axis_index("c")
        gbase = wid * _GROUPS_W
        pltpu.sync_copy(idx_hbm.at[pl.ds(gbase, _GROUPS_W)], idx_v)

        def body(i, carry):
            cps = [
                pltpu.async_copy(
                    table_hbm.at[idx_v.at[i * _SUPER + j]], buf.at[j], gsem)
                for j in range(_SUPER)
            ]
            for cp in cps:
                cp.wait()
            pltpu.sync_copy(buf, out_hbm.at[pl.ds(gbase + i * _SUPER, _SUPER)])
            return carry

        lax.fori_loop(0, _ITERS, body, 0)

    return gather_kernel


_gather = _build()


def kernel(idx, node_type_embed, degree_embed):
    idx0 = idx[:, :, 0].reshape(_N // _GROUP, _GROUP)
    out = _gather(node_type_embed, idx0)
    return out.reshape(_B, _L, _D)


# speedup vs baseline: 1.0010x; 1.0010x over previous
import functools

import jax
import jax.numpy as jnp
from jax import lax
from jax.experimental import pallas as pl
from jax.experimental.pallas import tpu as pltpu
from jax.experimental.pallas import tpu_sc as plsc

_B, _L, _D = 4096, 200, 64
_N = _B * _L
_NW = 32
_ROWS_W = _N // _NW
_CHUNK = 1024
_ITERS = _ROWS_W // _CHUNK


def _build():
    mesh = plsc.VectorSubcoreMesh(core_axis_name="c", subcore_axis_name="s")

    @functools.partial(
        pl.kernel,
        mesh=mesh,
        out_type=jax.ShapeDtypeStruct((_N, _D), jnp.float32),
        compiler_params=pltpu.CompilerParams(use_tc_tiling_on_sc=False),
        scratch_types=[
            pltpu.VMEM((_ROWS_W,), jnp.int32),
            pltpu.VMEM((_CHUNK, _D), jnp.float32),
            pltpu.SemaphoreType.DMA,
        ],
    )
    def gather_kernel(table_hbm, idx_hbm, out_hbm, idx_v, buf, gsem):
        wid = lax.axis_index("s") * 2 + lax.axis_index("c")
        rbase = wid * _ROWS_W
        pltpu.sync_copy(idx_hbm.at[pl.ds(rbase, _ROWS_W)], idx_v)

        def body(i, carry):
            pltpu.async_copy(
                table_hbm.at[idx_v.at[pl.ds(i * _CHUNK, _CHUNK)]], buf, gsem
            ).wait()
            pltpu.sync_copy(buf, out_hbm.at[pl.ds(rbase + i * _CHUNK, _CHUNK)])
            return carry

        lax.fori_loop(0, _ITERS, body, 0)

    return gather_kernel


_gather = _build()


def kernel(idx, node_type_embed, degree_embed):
    idx0 = idx[:, :, 0].reshape(_N)
    out = _gather(node_type_embed, idx0)
    return out.reshape(_B, _L, _D)


# speedup vs baseline: 10.6500x; 10.6390x over previous
import functools

import jax
import jax.numpy as jnp
from jax import lax
from jax.experimental import pallas as pl
from jax.experimental.pallas import tpu as pltpu
from jax.experimental.pallas import tpu_sc as plsc

_B, _L, _D = 4096, 200, 64
_N = _B * _L
_NW = 32
_ROWS_W = _N // _NW
_CHUNK = 1024
_ITERS = _ROWS_W // _CHUNK


def _build():
    mesh = plsc.VectorSubcoreMesh(core_axis_name="c", subcore_axis_name="s")

    @functools.partial(
        pl.kernel,
        mesh=mesh,
        out_type=jax.ShapeDtypeStruct((_N, _D), jnp.float32),
        compiler_params=pltpu.CompilerParams(use_tc_tiling_on_sc=False),
        scratch_types=[
            pltpu.VMEM((_ROWS_W,), jnp.int32),
            pltpu.VMEM((_CHUNK, _D), jnp.float32),
            pltpu.VMEM_SHARED((1000, _D), jnp.float32),
            pltpu.SemaphoreType.DMA,
        ],
    )
    def gather_kernel(table_hbm, idx_hbm, out_hbm, idx_v, buf, table_sp, gsem):
        sid = lax.axis_index("s")
        wid = sid * 2 + lax.axis_index("c")
        rbase = wid * _ROWS_W

        @pl.when(sid == 0)
        def _():
            pltpu.sync_copy(table_hbm, table_sp)

        pltpu.sync_copy(idx_hbm.at[pl.ds(rbase, _ROWS_W)], idx_v)
        plsc.subcore_barrier()

        def body(i, carry):
            pltpu.async_copy(
                table_sp.at[idx_v.at[pl.ds(i * _CHUNK, _CHUNK)]], buf, gsem
            ).wait()
            pltpu.sync_copy(buf, out_hbm.at[pl.ds(rbase + i * _CHUNK, _CHUNK)])
            return carry

        lax.fori_loop(0, _ITERS, body, 0)

    return gather_kernel


_gather = _build()


def kernel(idx, node_type_embed, degree_embed):
    idx0 = idx[:, :, 0].reshape(_N)
    out = _gather(node_type_embed, idx0)
    return out.reshape(_B, _L, _D)
